# baseline (speedup 1.0000x reference)
import functools

import numpy as np
import jax
import jax.numpy as jnp
from jax import lax
from jax.experimental import pallas as pl
from jax.experimental.pallas import tpu as pltpu
from jax.experimental.pallas import tpu_sc as plsc

_NC = 2
_NS = 16
_CH = 128

_D = 128
_A = 10
_R = 16
_EMB = 32
_MAX_DENSITY = 100.0


def _silu(x):
    return x * jax.nn.sigmoid(x)


def _edge_body(eft_ref, ea_ref, wr1, wr2, wr3, wr4, wd, w_out, dens_out):
    eft = eft_ref[...]
    h = _silu(lax.dot_general(eft, wr1[...], (((0,), (0,)), ((), ())),
                              preferred_element_type=jnp.float32) * 0.25)
    h = _silu(jnp.dot(h, wr2[...], preferred_element_type=jnp.float32) * 0.125)
    h = _silu(jnp.dot(h, wr3[...], preferred_element_type=jnp.float32) * 0.125)
    tp = jnp.dot(h, wr4[...], preferred_element_type=jnp.float32) * 0.125
    w_out[...] = tp * jnp.transpose(ea_ref[...])
    d = jnp.sum(eft * wd[...], axis=0) * 0.25
    be = eft.shape[1]
    off = pl.multiple_of(pl.program_id(0) * be, 128)
    dens_out[pl.ds(off, be)] = jnp.tanh(d * d)


def _edge_stage(eft, ea_row, wr1, wr2, wr3, wr4, wd, block_e, blk0, nblk):
    ep = nblk * block_e
    grid = nblk
    full = lambda shape: pl.BlockSpec(shape, lambda i: (0,) * len(shape))
    return pl.pallas_call(
        _edge_body,
        grid=(grid,),
        in_specs=[
            pl.BlockSpec((_R, block_e), lambda i: (0, i + blk0)),
            pl.BlockSpec((1, block_e), lambda i: (0, i + blk0)),
            full((_R, 64)), full((64, 64)), full((64, 64)), full((64, _D)),
            full((_R, 1)),
        ],
        out_specs=[
            pl.BlockSpec((block_e, _D), lambda i: (i, 0)),
            pl.BlockSpec((ep,), lambda i: (0,)),
        ],
        out_shape=[
            jax.ShapeDtypeStruct((ep, _D), jnp.float32),
            jax.ShapeDtypeStruct((ep,), jnp.float32),
        ],
    )(eft, ea_row, wr1, wr2, wr3, wr4, wd)


def _nf_body(x_ref, wup, nf_out):
    nf_out[...] = jnp.dot(x_ref[...], wup[...],
                          preferred_element_type=jnp.float32) * (
        1.0 / np.sqrt(_D))


def _nf_stage(node_feats, wup, block_n):
    n = node_feats.shape[0]
    full = lambda shape: pl.BlockSpec(shape, lambda i: (0,) * len(shape))
    return pl.pallas_call(
        _nf_body,
        grid=(n // block_n,),
        in_specs=[
            pl.BlockSpec((block_n, _D), lambda i: (i, 0)),
            full((_D, _D)),
        ],
        out_specs=pl.BlockSpec((block_n, _D), lambda i: (i, 0)),
        out_shape=jax.ShapeDtypeStruct((n, _D), jnp.float32),
    )(node_feats, wup)


def _skip_body(x_ref, a_ref, wskip, sc_out):
    x = x_ref[...]
    a = a_ref[...]
    acc = jnp.zeros(x.shape, jnp.float32)
    for j in range(_A):
        acc = acc + a[:, j:j + 1] * jnp.dot(
            x, wskip[:, j, :], preferred_element_type=jnp.float32)
    sc_out[...] = acc * (1.0 / np.sqrt(_D * _A))


def _skip_stage(node_feats, node_attrs, wskip, block_n):
    n = node_feats.shape[0]
    full = lambda shape: pl.BlockSpec(shape, lambda i: (0,) * len(shape))
    return pl.pallas_call(
        _skip_body,
        grid=(n // block_n,),
        in_specs=[
            pl.BlockSpec((block_n, _D), lambda i: (i, 0)),
            pl.BlockSpec((block_n, _A), lambda i: (i, 0)),
            full((_D, _A, _D)),
        ],
        out_specs=pl.BlockSpec((block_n, _D), lambda i: (i, 0)),
        out_shape=jax.ShapeDtypeStruct((n, _D), jnp.float32),
    )(node_feats, node_attrs, wskip)


def _make_sc_scatter(n_msg, n_dens, ep, chunk, nb=2):
    per_tile = ep // (_NC * _NS)
    n_chunks = per_tile // chunk
    mrows = n_msg // _NS
    drows = n_dens // _NS
    zrows = 8
    mesh = plsc.VectorSubcoreMesh(core_axis_name="c", subcore_axis_name="s")

    @functools.partial(
        pl.kernel,
        mesh=mesh,
        out_type=(
            jax.ShapeDtypeStruct((_NC, n_msg, _D), jnp.float32),
            jax.ShapeDtypeStruct((_NC, n_dens), jnp.float32),
        ),
        scratch_types=[
            [pltpu.VMEM((chunk,), jnp.int32)] * nb,
            [pltpu.VMEM((chunk,), jnp.int32)] * nb,
            [pltpu.VMEM((chunk, _D), jnp.float32)] * nb,
            [pltpu.VMEM((chunk, _D), jnp.float32)] * nb,
            [pltpu.VMEM((chunk,), jnp.float32)] * nb,
            pltpu.VMEM((zrows, _D), jnp.float32),
            pltpu.VMEM((drows,), jnp.float32),
            pltpu.VMEM_SHARED((n_msg, _D), jnp.float32),
            pltpu.VMEM_SHARED((n_dens,), jnp.float32),
            [pltpu.SemaphoreType.DMA] * nb,
            [pltpu.SemaphoreType.DMA] * nb,
            [pltpu.SemaphoreType.DMA] * nb,
        ],
    )
    def sc_kernel(nf, w, dens, snd, rcv, msg_out, dens_out,
                  sidx, ridx, wbuf, rows, dbuf, z128, z1,
                  msg_acc, dens_acc, semg, semi, sems):
        c = lax.axis_index("c")
        s = lax.axis_index("s")
        tile = c * _NS + s
        zvec = jnp.zeros((16,), jnp.float32)

        def zero_z(i, carry):
            for k in range(_D // 16):
                z128[i, pl.ds(k * 16, 16)] = zvec
            return carry
        lax.fori_loop(0, zrows, zero_z, 0)

        def zero_d(i, carry):
            z1[pl.ds(i * 16, 16)] = zvec
            return carry
        lax.fori_loop(0, drows // 16, zero_d, 0)

        for b in range(mrows // zrows):
            r0 = s * mrows + b * zrows
            pltpu.sync_copy(z128, msg_acc.at[pl.ds(r0, zrows)])
        pltpu.sync_copy(z1, dens_acc.at[pl.ds(s * drows, drows)])
        plsc.subcore_barrier()

        base = tile * per_tile

        def stage(i, b):
            off = base + i * chunk
            pltpu.sync_copy(snd.at[pl.ds(off, chunk)], sidx[b])
            pltpu.async_copy(nf.at[sidx[b]], rows[b], semg[b])
            pltpu.async_copy(rcv.at[pl.ds(off, chunk)], ridx[b], semi[b])
            pltpu.async_copy(w.at[pl.ds(off, chunk)], wbuf[b], semi[b])
            pltpu.async_copy(dens.at[pl.ds(off, chunk)], dbuf[b], semi[b])

        def wait_in(i, b):
            off = base + i * chunk
            pltpu.make_async_copy(nf.at[sidx[b]], rows[b], semg[b]).wait()
            pltpu.make_async_copy(rcv.at[pl.ds(off, chunk)], ridx[b],
                                  semi[b]).wait()
            pltpu.make_async_copy(w.at[pl.ds(off, chunk)], wbuf[b],
                                  semi[b]).wait()
            pltpu.make_async_copy(dens.at[pl.ds(off, chunk)], dbuf[b],
                                  semi[b]).wait()

        def mult(b):
            @plsc.parallel_loop(0, chunk, 1, unroll=4)
            def _(j):
                for k in range(_D // 16):
                    sl = pl.ds(k * 16, 16)
                    rows[b][j, sl] = rows[b][j, sl] * wbuf[b][j, sl]

        def issue_scatter(b):
            pltpu.async_copy(rows[b], msg_acc.at[ridx[b]], sems[b], add=True)
            pltpu.async_copy(dbuf[b], dens_acc.at[ridx[b]], sems[b], add=True)

        def drain_scatter(b):
            pltpu.make_async_copy(rows[b], msg_acc.at[ridx[b]],
                                  sems[b]).wait()
            pltpu.make_async_copy(dbuf[b], dens_acc.at[ridx[b]],
                                  sems[b]).wait()

        def step(i, b, first):
            bs = (b + nb - 1) % nb
            if not first:
                drain_scatter(bs)

            @pl.when(i + nb - 1 < n_chunks)
            def _():
                stage(i + nb - 1, bs)
            wait_in(i, b)
            mult(b)
            issue_scatter(b)

        for j in range(nb - 1):
            stage(j, j)
        step(0, 0, first=True)

        def round_body(r, carry):
            i = 1 + nb * r
            for s in range(nb):
                step(i + s, (1 + s) % nb, first=False)
            return carry
        q = (n_chunks - 1) // nb
        lax.fori_loop(0, q, round_body, 0)
        for i in range(nb * q + 1, n_chunks):
            step(i, i % nb, first=False)
        drain_scatter((n_chunks - 1) % nb)

        plsc.subcore_barrier()
        pltpu.sync_copy(msg_acc.at[pl.ds(s * mrows, mrows)],
                        msg_out.at[c, pl.ds(s * mrows, mrows)])
        pltpu.sync_copy(dens_acc.at[pl.ds(s * drows, drows)],
                        dens_out.at[c, pl.ds(s * drows, drows)])

    return sc_kernel


def _final_body(msgp_ref, msgq_ref, msgr_ref, msgs_ref, densp_ref,
                densq_ref, densr_ref, denss_ref, wdlin, wlin, out_ref):
    m = (msgp_ref[0] + msgp_ref[1] + msgq_ref[0] + msgq_ref[1]
         + msgr_ref[0] + msgr_ref[1] + msgs_ref[0] + msgs_ref[1])
    nd = (densp_ref[:, 0:1] + densp_ref[:, 1:2]
          + densq_ref[:, 0:1] + densq_ref[:, 1:2]
          + densr_ref[:, 0:1] + densr_ref[:, 1:2]
          + denss_ref[:, 0:1] + denss_ref[:, 1:2])
    inv = 1.0 / (nd + 1.0)
    m = m * inv
    half = _EMB // 2
    freqs = jnp.exp(
        lax.broadcasted_iota(jnp.int32, (1, half), 1).astype(jnp.float32)
        * (-np.log(_MAX_DENSITY) / half))
    args = nd * freqs
    emb = (jnp.dot(jnp.sin(args), wdlin[0:_EMB // 2, :],
                   preferred_element_type=jnp.float32)
           + jnp.dot(jnp.cos(args), wdlin[_EMB // 2:, :],
                     preferred_element_type=jnp.float32))
    m = m + emb
    out_ref[...] = jnp.dot(m, wlin[...], preferred_element_type=jnp.float32) * (
        1.0 / np.sqrt(_D)) * inv


def _final_stage(msgs, denss, wdlin, wlin, n, block_n):
    grid = n // block_n
    full = lambda shape: pl.BlockSpec(shape, lambda i: (0,) * len(shape))
    return pl.pallas_call(
        _final_body,
        grid=(grid,),
        in_specs=(
            [pl.BlockSpec((_NC, block_n, _D), lambda i: (0, i, 0))] * 4
            + [pl.BlockSpec((block_n, _NC), lambda i: (i, 0))] * 4
            + [full((_EMB, _D)), full((_D, _D))]
        ),
        out_specs=pl.BlockSpec((block_n, _D), lambda i: (i, 0)),
        out_shape=jax.ShapeDtypeStruct((n, _D), jnp.float32),
    )(*msgs, *denss, wdlin, wlin)


def kernel(node_attrs, node_feats, edge_attrs, edge_feats, edge_index,
           W_up, W_r1, W_r2, W_r3, W_r4, W_skip, W_lin, W_density, W_dlin):
    n = node_feats.shape[0]
    e = edge_feats.shape[0]
    block_e = 2560
    blocks = e // block_e
    nslice = 4
    bs_list = [blocks // nslice] * (nslice - 1)
    bs_list.append(blocks - sum(bs_list))

    nf = _nf_stage(node_feats, W_up, block_n=1000)
    eft = edge_feats.T
    eat = edge_attrs.T
    sender = edge_index[0]
    receiver = edge_index[1]

    n_msg = ((n + _NS * 8 - 1) // (_NS * 8)) * (_NS * 8)
    n_dens = ((n + _NS * 16 - 1) // (_NS * 16)) * (_NS * 16)

    msgs, denss = [], []
    blk0 = 0
    for bsl in bs_list:
        esl = bsl * block_e
        off = blk0 * block_e
        wsl, dsl = _edge_stage(eft, eat, W_r1, W_r2, W_r3, W_r4,
                               W_density, block_e, blk0, bsl)
        scat = _make_sc_scatter(n_msg, n_dens, esl, chunk=80)
        m, dp = scat(nf, wsl, dsl, sender[off:off + esl],
                     receiver[off:off + esl])
        msgs.append(m)
        denss.append(dp.T)
        blk0 += bsl

    sc = _skip_stage(node_feats, node_attrs, W_skip, block_n=1000)

    out = _final_stage(msgs, denss, W_dlin, W_lin, n, block_n=1000)
    return out.reshape(n, _D, 1), sc


# speedup vs baseline: 1.0291x; 1.0291x over previous
import functools

import numpy as np
import jax
import jax.numpy as jnp
from jax import lax
from jax.experimental import pallas as pl
from jax.experimental.pallas import tpu as pltpu
from jax.experimental.pallas import tpu_sc as plsc

_NC = 2
_NS = 16
_CH = 128

_D = 128
_A = 10
_R = 16
_EMB = 32
_MAX_DENSITY = 100.0


def _silu(x):
    return x * jax.nn.sigmoid(x)


def _edge_body(eft_ref, ea_ref, wr1, wr2, wr3, wr4, wd, w_out, dens_out):
    eft = eft_ref[...]
    h = _silu(lax.dot_general(eft, wr1[...], (((0,), (0,)), ((), ())),
                              preferred_element_type=jnp.float32) * 0.25)
    h = _silu(jnp.dot(h, wr2[...], preferred_element_type=jnp.float32) * 0.125)
    h = _silu(jnp.dot(h, wr3[...], preferred_element_type=jnp.float32) * 0.125)
    tp = jnp.dot(h, wr4[...], preferred_element_type=jnp.float32) * 0.125
    w_out[...] = tp * jnp.transpose(ea_ref[...])
    d = jnp.sum(eft * wd[...], axis=0) * 0.25
    be = eft.shape[1]
    off = pl.multiple_of(pl.program_id(0) * be, 128)
    dens_out[pl.ds(off, be)] = jnp.tanh(d * d)


def _edge_stage(eft, ea_row, wr1, wr2, wr3, wr4, wd, block_e, blk0, nblk):
    ep = nblk * block_e
    grid = nblk
    full = lambda shape: pl.BlockSpec(shape, lambda i: (0,) * len(shape))
    return pl.pallas_call(
        _edge_body,
        grid=(grid,),
        in_specs=[
            pl.BlockSpec((_R, block_e), lambda i: (0, i + blk0)),
            pl.BlockSpec((1, block_e), lambda i: (0, i + blk0)),
            full((_R, 64)), full((64, 64)), full((64, 64)), full((64, _D)),
            full((_R, 1)),
        ],
        out_specs=[
            pl.BlockSpec((block_e, _D), lambda i: (i, 0)),
            pl.BlockSpec((ep,), lambda i: (0,)),
        ],
        out_shape=[
            jax.ShapeDtypeStruct((ep, _D), jnp.float32),
            jax.ShapeDtypeStruct((ep,), jnp.float32),
        ],
    )(eft, ea_row, wr1, wr2, wr3, wr4, wd)


def _nf_body(x_ref, wup, nf_out):
    nf_out[...] = jnp.dot(x_ref[...], wup[...],
                          preferred_element_type=jnp.float32) * (
        1.0 / np.sqrt(_D))


def _nf_stage(node_feats, wup, block_n):
    n = node_feats.shape[0]
    full = lambda shape: pl.BlockSpec(shape, lambda i: (0,) * len(shape))
    return pl.pallas_call(
        _nf_body,
        grid=(n // block_n,),
        in_specs=[
            pl.BlockSpec((block_n, _D), lambda i: (i, 0)),
            full((_D, _D)),
        ],
        out_specs=pl.BlockSpec((block_n, _D), lambda i: (i, 0)),
        out_shape=jax.ShapeDtypeStruct((n, _D), jnp.float32),
    )(node_feats, wup)


def _skip_body(x_ref, a_ref, wskip, sc_out):
    x = x_ref[...]
    a = a_ref[...]
    acc = jnp.zeros(x.shape, jnp.float32)
    for j in range(_A):
        acc = acc + a[:, j:j + 1] * jnp.dot(
            x, wskip[:, j, :], preferred_element_type=jnp.float32)
    sc_out[...] = acc * (1.0 / np.sqrt(_D * _A))


def _skip_stage(node_feats, node_attrs, wskip, block_n):
    n = node_feats.shape[0]
    full = lambda shape: pl.BlockSpec(shape, lambda i: (0,) * len(shape))
    return pl.pallas_call(
        _skip_body,
        grid=(n // block_n,),
        in_specs=[
            pl.BlockSpec((block_n, _D), lambda i: (i, 0)),
            pl.BlockSpec((block_n, _A), lambda i: (i, 0)),
            full((_D, _A, _D)),
        ],
        out_specs=pl.BlockSpec((block_n, _D), lambda i: (i, 0)),
        out_shape=jax.ShapeDtypeStruct((n, _D), jnp.float32),
    )(node_feats, node_attrs, wskip)


def _make_sc_scatter(n_msg, n_dens, ep, chunk, nb=2):
    per_tile = ep // (_NC * _NS)
    n_chunks = per_tile // chunk
    mrows = n_msg // _NS
    drows = n_dens // _NS
    zrows = 8
    mesh = plsc.VectorSubcoreMesh(core_axis_name="c", subcore_axis_name="s")

    @functools.partial(
        pl.kernel,
        mesh=mesh,
        out_type=(
            jax.ShapeDtypeStruct((_NC, n_msg, _D), jnp.float32),
            jax.ShapeDtypeStruct((_NC, n_dens), jnp.float32),
        ),
        scratch_types=[
            [pltpu.VMEM((chunk,), jnp.int32)] * nb,
            [pltpu.VMEM((chunk,), jnp.int32)] * nb,
            [pltpu.VMEM((chunk, _D), jnp.float32)] * nb,
            [pltpu.VMEM((chunk, _D), jnp.float32)] * nb,
            [pltpu.VMEM((chunk,), jnp.float32)] * nb,
            pltpu.VMEM((zrows, _D), jnp.float32),
            pltpu.VMEM((drows,), jnp.float32),
            pltpu.VMEM_SHARED((n_msg, _D), jnp.float32),
            pltpu.VMEM_SHARED((n_dens,), jnp.float32),
            [pltpu.SemaphoreType.DMA] * nb,
            [pltpu.SemaphoreType.DMA] * nb,
            [pltpu.SemaphoreType.DMA] * nb,
        ],
    )
    def sc_kernel(nf, w, dens, snd, rcv, msg_out, dens_out,
                  sidx, ridx, wbuf, rows, dbuf, z128, z1,
                  msg_acc, dens_acc, semg, semi, sems):
        c = lax.axis_index("c")
        s = lax.axis_index("s")
        tile = c * _NS + s
        zvec = jnp.zeros((16,), jnp.float32)

        def zero_z(i, carry):
            for k in range(_D // 16):
                z128[i, pl.ds(k * 16, 16)] = zvec
            return carry
        lax.fori_loop(0, zrows, zero_z, 0)

        def zero_d(i, carry):
            z1[pl.ds(i * 16, 16)] = zvec
            return carry
        lax.fori_loop(0, drows // 16, zero_d, 0)

        for b in range(mrows // zrows):
            r0 = s * mrows + b * zrows
            pltpu.sync_copy(z128, msg_acc.at[pl.ds(r0, zrows)])
        pltpu.sync_copy(z1, dens_acc.at[pl.ds(s * drows, drows)])
        plsc.subcore_barrier()

        base = tile * per_tile

        def stage(i, b):
            off = base + i * chunk
            pltpu.sync_copy(snd.at[pl.ds(off, chunk)], sidx[b])
            pltpu.async_copy(nf.at[sidx[b]], rows[b], semg[b])
            pltpu.async_copy(rcv.at[pl.ds(off, chunk)], ridx[b], semi[b])
            pltpu.async_copy(w.at[pl.ds(off, chunk)], wbuf[b], semi[b])
            pltpu.async_copy(dens.at[pl.ds(off, chunk)], dbuf[b], semi[b])

        def wait_in(i, b):
            off = base + i * chunk
            pltpu.make_async_copy(nf.at[sidx[b]], rows[b], semg[b]).wait()
            pltpu.make_async_copy(rcv.at[pl.ds(off, chunk)], ridx[b],
                                  semi[b]).wait()
            pltpu.make_async_copy(w.at[pl.ds(off, chunk)], wbuf[b],
                                  semi[b]).wait()
            pltpu.make_async_copy(dens.at[pl.ds(off, chunk)], dbuf[b],
                                  semi[b]).wait()

        def mult(b):
            @plsc.parallel_loop(0, chunk, 1, unroll=4)
            def _(j):
                for k in range(_D // 16):
                    sl = pl.ds(k * 16, 16)
                    rows[b][j, sl] = rows[b][j, sl] * wbuf[b][j, sl]

        def issue_scatter(b):
            pltpu.async_copy(rows[b], msg_acc.at[ridx[b]], sems[b], add=True)
            pltpu.async_copy(dbuf[b], dens_acc.at[ridx[b]], sems[b], add=True)

        def drain_scatter(b):
            pltpu.make_async_copy(rows[b], msg_acc.at[ridx[b]],
                                  sems[b]).wait()
            pltpu.make_async_copy(dbuf[b], dens_acc.at[ridx[b]],
                                  sems[b]).wait()

        def step(i, b, first):
            bs = (b + nb - 1) % nb
            if not first:
                drain_scatter(bs)

            @pl.when(i + nb - 1 < n_chunks)
            def _():
                stage(i + nb - 1, bs)
            wait_in(i, b)
            mult(b)
            issue_scatter(b)

        for j in range(nb - 1):
            stage(j, j)
        step(0, 0, first=True)

        def round_body(r, carry):
            i = 1 + nb * r
            for s in range(nb):
                step(i + s, (1 + s) % nb, first=False)
            return carry
        q = (n_chunks - 1) // nb
        lax.fori_loop(0, q, round_body, 0)
        for i in range(nb * q + 1, n_chunks):
            step(i, i % nb, first=False)
        drain_scatter((n_chunks - 1) % nb)

        plsc.subcore_barrier()
        pltpu.sync_copy(msg_acc.at[pl.ds(s * mrows, mrows)],
                        msg_out.at[c, pl.ds(s * mrows, mrows)])
        pltpu.sync_copy(dens_acc.at[pl.ds(s * drows, drows)],
                        dens_out.at[c, pl.ds(s * drows, drows)])

    return sc_kernel


def _final_body(msgp_ref, msgq_ref, msgr_ref, densp_ref, densq_ref,
                densr_ref, wdlin, wlin, out_ref):
    m = (msgp_ref[0] + msgp_ref[1] + msgq_ref[0] + msgq_ref[1]
         + msgr_ref[0] + msgr_ref[1])
    nd = (densp_ref[:, 0:1] + densp_ref[:, 1:2]
          + densq_ref[:, 0:1] + densq_ref[:, 1:2]
          + densr_ref[:, 0:1] + densr_ref[:, 1:2])
    inv = 1.0 / (nd + 1.0)
    m = m * inv
    half = _EMB // 2
    freqs = jnp.exp(
        lax.broadcasted_iota(jnp.int32, (1, half), 1).astype(jnp.float32)
        * (-np.log(_MAX_DENSITY) / half))
    args = nd * freqs
    emb = (jnp.dot(jnp.sin(args), wdlin[0:_EMB // 2, :],
                   preferred_element_type=jnp.float32)
           + jnp.dot(jnp.cos(args), wdlin[_EMB // 2:, :],
                     preferred_element_type=jnp.float32))
    m = m + emb
    out_ref[...] = jnp.dot(m, wlin[...], preferred_element_type=jnp.float32) * (
        1.0 / np.sqrt(_D)) * inv


def _final_stage(msgs, denss, wdlin, wlin, n, block_n):
    grid = n // block_n
    full = lambda shape: pl.BlockSpec(shape, lambda i: (0,) * len(shape))
    return pl.pallas_call(
        _final_body,
        grid=(grid,),
        in_specs=(
            [pl.BlockSpec((_NC, block_n, _D), lambda i: (0, i, 0))] * 3
            + [pl.BlockSpec((block_n, _NC), lambda i: (i, 0))] * 3
            + [full((_EMB, _D)), full((_D, _D))]
        ),
        out_specs=pl.BlockSpec((block_n, _D), lambda i: (i, 0)),
        out_shape=jax.ShapeDtypeStruct((n, _D), jnp.float32),
    )(*msgs, *denss, wdlin, wlin)


def kernel(node_attrs, node_feats, edge_attrs, edge_feats, edge_index,
           W_up, W_r1, W_r2, W_r3, W_r4, W_skip, W_lin, W_density, W_dlin):
    n = node_feats.shape[0]
    e = edge_feats.shape[0]
    block_e = 2560
    blocks = e // block_e
    nslice = 3
    bs_list = [blocks // nslice] * (nslice - 1)
    bs_list.append(blocks - sum(bs_list))

    nf = _nf_stage(node_feats, W_up, block_n=1000)
    eft = edge_feats.T
    eat = edge_attrs.T
    sender = edge_index[0]
    receiver = edge_index[1]

    n_msg = ((n + _NS * 8 - 1) // (_NS * 8)) * (_NS * 8)
    n_dens = ((n + _NS * 16 - 1) // (_NS * 16)) * (_NS * 16)

    msgs, denss = [], []
    blk0 = 0
    for bsl in bs_list:
        esl = bsl * block_e
        off = blk0 * block_e
        wsl, dsl = _edge_stage(eft, eat, W_r1, W_r2, W_r3, W_r4,
                               W_density, block_e, blk0, bsl)
        scat = _make_sc_scatter(n_msg, n_dens, esl, chunk=80)
        m, dp = scat(nf, wsl, dsl, sender[off:off + esl],
                     receiver[off:off + esl])
        msgs.append(m)
        denss.append(dp.T)
        blk0 += bsl

    sc = _skip_stage(node_feats, node_attrs, W_skip, block_n=1000)

    out = _final_stage(msgs, denss, W_dlin, W_lin, n, block_n=1000)
    return out.reshape(n, _D, 1), sc
